# baseline (speedup 1.0000x reference)
import functools

import jax
import jax.numpy as jnp
from jax import lax
from jax.experimental import pallas as pl
from jax.experimental.pallas import tpu as pltpu
from jax.experimental.pallas import tpu_sc as plsc

N = 10000
DF = 128
H = 32
E = 640000
B = 1024
EPS = 1e-3

NC, NS, L = 2, 16, 16
NW = NC * NS
CHUNK = 128
E_PAD = 655360
E_PER_W = E_PAD // NW
N_CHUNKS = E_PER_W // CHUNK
N_PAD = 10112
ROWS_PER_TILE = N_PAD // NS

_SQRT2 = 1.4142135623730951


def _gelu(x):
    return 0.5 * x * (1.0 + lax.erf(x / _SQRT2))


def _gstage(x, s, t, w, b):
    return _gelu(jnp.dot(x * s + t, w, preferred_element_type=jnp.float32) + b)


def _l2norm(x):
    return x * lax.rsqrt(jnp.maximum(jnp.sum(x * x, axis=-1, keepdims=True), 1e-12))


def _fold(layer):
    s = layer["gamma"] * lax.rsqrt(layer["var"] + EPS)
    t = layer["beta"] - layer["mean"] * s
    return s.reshape(1, -1), t.reshape(1, -1), layer["W"], layer["b"].reshape(1, -1)


def _tc1(nf, ew2d, pre, prep1):
    def body(nf_r, ew_r,
             s0, t0, w0, b0, s1, t1, w1, b1,
             q0, r0, v0, c0, q1, r1, v1, c1,
             x0_o, p1_o, inv_o):
        x = nf_r[...]
        x = _gstage(x, s0[...], t0[...], w0[...], b0[...])
        x = _gstage(x, s1[...], t1[...], w1[...], b1[...])
        x0_o[...] = x
        p = _gstage(x, q0[...], r0[...], v0[...], c0[...])
        p = _gstage(p, q1[...], r1[...], v1[...], c1[...])
        p1_o[...] = p
        inv_o[...] = (1.0 / jnp.sum(ew_r[...])).reshape(1, 1)

    outs = [jax.ShapeDtypeStruct((N, H), jnp.float32),
            jax.ShapeDtypeStruct((N, H), jnp.float32),
            jax.ShapeDtypeStruct((1, 1), jnp.float32)]
    return pl.pallas_call(body, out_shape=outs)(
        nf, ew2d, *pre[0], *pre[1], *prep1[0], *prep1[1])


def _tc_update(x, pa, pb, inv_s, upd, nxt):
    (s0, t0, w0, b0), (s1, t1, w1, b1) = upd
    sxa, sxb = s0[:, :H], s0[:, H:]
    txa, txb = t0[:, :H], t0[:, H:]
    wa, wb = w0[:H], w0[H:]
    (m0s, m0t, m0w, m0b), (m1s, m1t, m1w, m1b) = nxt

    def body(x_r, pa_r, pb_r, inv_r,
             sxa_r, txa_r, sxb_r, txb_r, wa_r, wb_r, b0_r,
             s1_r, t1_r, w1_r, b1_r,
             a0s, a0t, a0w, a0b, a1s, a1t, a1w, a1b,
             xo, po):
        xx = x_r[...]
        agg = (pa_r[...] + pb_r[...]) * inv_r[0, 0]
        h = (jnp.dot(xx * sxa_r[...] + txa_r[...], wa_r[...],
                     preferred_element_type=jnp.float32)
             + jnp.dot(agg * sxb_r[...] + txb_r[...], wb_r[...],
                       preferred_element_type=jnp.float32)
             + b0_r[...])
        u = _gelu(h)
        u = _gstage(u, s1_r[...], t1_r[...], w1_r[...], b1_r[...])
        x_new = _l2norm(u) + xx
        xo[...] = x_new
        p = _gstage(x_new, a0s[...], a0t[...], a0w[...], a0b[...])
        p = _gstage(p, a1s[...], a1t[...], a1w[...], a1b[...])
        po[...] = p

    outs = [jax.ShapeDtypeStruct((N, H), jnp.float32),
            jax.ShapeDtypeStruct((N, H), jnp.float32)]
    return pl.pallas_call(body, out_shape=outs)(
        x, pa, pb, inv_s, sxa, txa, sxb, txb, wa, wb, b0,
        s1, t1, w1, b1, m0s, m0t, m0w, m0b, m1s, m1t, m1w, m1b)


def _tc_logits(emb, w, b):
    def body(e_r, w_r, b_r, o):
        o[...] = jnp.dot(e_r[...], w_r[...],
                         preferred_element_type=jnp.float32) + b_r[...]

    return pl.pallas_call(
        body, out_shape=jax.ShapeDtypeStruct((B, w.shape[1]), jnp.float32),
    )(emb, w, b)


def _sc_aggregate(prep, src2d, dst2d, ew2d, zrows):
    mesh = plsc.VectorSubcoreMesh(core_axis_name="c", subcore_axis_name="s")

    @functools.partial(
        pl.kernel,
        mesh=mesh,
        compiler_params=pltpu.CompilerParams(use_tc_tiling_on_sc=False),
        out_type=jax.ShapeDtypeStruct((NC, N_PAD, H), jnp.float32),
        scratch_types=[
            pltpu.VMEM((N_CHUNKS, CHUNK), jnp.int32),
            pltpu.VMEM((N_CHUNKS, CHUNK), jnp.int32),
            pltpu.VMEM((N_CHUNKS, CHUNK), jnp.float32),
            pltpu.VMEM((CHUNK, H), jnp.float32),
            pltpu.VMEM((CHUNK, H), jnp.float32),
            pltpu.VMEM_SHARED((N_PAD, H), jnp.float32),
            pltpu.SemaphoreType.DMA,
            pltpu.SemaphoreType.DMA,
            pltpu.SemaphoreType.DMA,
            pltpu.SemaphoreType.DMA,
        ],
    )
    def k(prep_hbm, src_hbm, dst_hbm, ew_hbm, z_hbm, out_hbm,
          src_v, dst_v, ew_v, rows0_v, rows1_v, agg_sh, g0, g1, s0, s1):
        c = lax.axis_index("c")
        s = lax.axis_index("s")
        w = c * NS + s
        pltpu.sync_copy(z_hbm, agg_sh.at[pl.ds(s * ROWS_PER_TILE, ROWS_PER_TILE)])
        pltpu.sync_copy(src_hbm.at[pl.ds(w * N_CHUNKS, N_CHUNKS)], src_v)
        pltpu.sync_copy(dst_hbm.at[pl.ds(w * N_CHUNKS, N_CHUNKS)], dst_v)
        pltpu.sync_copy(ew_hbm.at[pl.ds(w * N_CHUNKS, N_CHUNKS)], ew_v)
        plsc.subcore_barrier()

        def scale(rows_v, t):
            for g in range(CHUNK // L):
                ewg = ew_v[t, pl.ds(g * L, L)]
                for j in range(L):
                    e = g * L + j
                    splat = lax.gather(
                        ewg, jnp.full((L, 1), j, jnp.int32),
                        lax.GatherDimensionNumbers(
                            offset_dims=(), collapsed_slice_dims=(0,),
                            start_index_map=(0,)),
                        slice_sizes=(1,),
                        mode=lax.GatherScatterMode.PROMISE_IN_BOUNDS)
                    rows_v[e, pl.ds(0, L)] = rows_v[e, pl.ds(0, L)] * splat
                    rows_v[e, pl.ds(L, L)] = rows_v[e, pl.ds(L, L)] * splat

        def start_gather(t, buf, sem):
            pltpu.async_copy(prep_hbm.at[src_v.at[t]], buf, sem)

        def wait_gather(t, buf, sem):
            pltpu.make_async_copy(prep_hbm.at[src_v.at[t]], buf, sem).wait()

        start_gather(0, rows0_v, g0)

        def pair_body(i, carry):
            t0 = 2 * i
            t1 = t0 + 1
            start_gather(t1, rows1_v, g1)
            wait_gather(t0, rows0_v, g0)
            scale(rows0_v, t0)
            h0 = pltpu.async_copy(rows0_v, agg_sh.at[dst_v.at[t0]], s0,
                                  add=True)
            wait_gather(t1, rows1_v, g1)
            scale(rows1_v, t1)
            h1 = pltpu.async_copy(rows1_v, agg_sh.at[dst_v.at[t1]], s1,
                                  add=True)
            h0.wait()

            @pl.when(t0 + 2 < N_CHUNKS)
            def _():
                start_gather(t0 + 2, rows0_v, g0)

            h1.wait()
            return carry

        lax.fori_loop(0, N_CHUNKS // 2, pair_body, 0)
        plsc.subcore_barrier()
        pltpu.sync_copy(
            agg_sh.at[pl.ds(s * ROWS_PER_TILE, ROWS_PER_TILE)],
            out_hbm.at[c, pl.ds(s * ROWS_PER_TILE, ROWS_PER_TILE)])

    return k(prep, src2d, dst2d, ew2d, zrows)


def _sc_gather_rows(table, idx):
    b_per_w = B // NW
    mesh = plsc.VectorSubcoreMesh(core_axis_name="c", subcore_axis_name="s")

    @functools.partial(
        pl.kernel,
        mesh=mesh,
        compiler_params=pltpu.CompilerParams(use_tc_tiling_on_sc=False),
        out_type=jax.ShapeDtypeStruct((B, H), jnp.float32),
        scratch_types=[
            pltpu.VMEM((b_per_w,), jnp.int32),
            pltpu.VMEM((b_per_w, H), jnp.float32),
            pltpu.SemaphoreType.DMA,
        ],
    )
    def k(table_hbm, idx_hbm, out_hbm, idx_v, rows_v, sem):
        w = lax.axis_index("s") * NC + lax.axis_index("c")
        base = w * b_per_w
        pltpu.sync_copy(idx_hbm.at[pl.ds(base, b_per_w)], idx_v)
        pltpu.async_copy(table_hbm.at[idx_v], rows_v, sem).wait()
        pltpu.sync_copy(rows_v, out_hbm.at[pl.ds(base, b_per_w)])

    return k(table, idx)


def kernel(node_features, edge_weights, params, edges, input_node_indices):
    p = params
    pre = [_fold(l) for l in p["preprocess"]]
    prep1 = [_fold(l) for l in p["conv1_prepare"]]
    upd1 = [_fold(l) for l in p["conv1_update"]]
    prep2 = [_fold(l) for l in p["conv2_prepare"]]
    upd2 = [_fold(l) for l in p["conv2_update"]]
    post = [_fold(l) for l in p["postprocess"]]

    ew_sum2d = edge_weights.reshape(E // DF, DF)
    zpad_i = jnp.zeros((E_PAD - E,), jnp.int32)
    zpad_f = jnp.zeros((E_PAD - E,), jnp.float32)
    dpad_i = jnp.arange(E_PAD - E, dtype=jnp.int32) % N_PAD
    src2d = jnp.concatenate([edges[1], zpad_i]).reshape(E_PAD // CHUNK, CHUNK)
    dst2d = jnp.concatenate([edges[0], dpad_i]).reshape(E_PAD // CHUNK, CHUNK)
    ew2d = jnp.concatenate([edge_weights, zpad_f]).reshape(E_PAD // CHUNK, CHUNK)

    zrows = jnp.zeros((ROWS_PER_TILE, H), jnp.float32)

    x0, prep1_t, inv_s = _tc1(node_features, ew_sum2d, pre, prep1)
    part1 = _sc_aggregate(prep1_t, src2d, dst2d, ew2d, zrows)
    x1, prep2_t = _tc_update(x0, part1[0, :N], part1[1, :N], inv_s, upd1, prep2)
    part2 = _sc_aggregate(prep2_t, src2d, dst2d, ew2d, zrows)
    _, y = _tc_update(x1, part2[0, :N], part2[1, :N], inv_s, upd2, post)
    emb = _sc_gather_rows(y, input_node_indices)
    return _tc_logits(emb, p["logits_W"], p["logits_b"].reshape(1, -1))


# speedup vs baseline: 1.6485x; 1.6485x over previous
import functools

import jax
import jax.numpy as jnp
from jax import lax
from jax.experimental import pallas as pl
from jax.experimental.pallas import tpu as pltpu
from jax.experimental.pallas import tpu_sc as plsc

N = 10000
DF = 128
H = 32
E = 640000
B = 1024
EPS = 1e-3

NC, NS, L = 2, 16, 16
NW = NC * NS
CHUNK = 80
E_PER_W = E // NW
N_CHUNKS = E_PER_W // CHUNK
N_PAD = 10112
ROWS_PER_TILE = N_PAD // NS

_SQRT2 = 1.4142135623730951


def _gelu(x):
    return 0.5 * x * (1.0 + lax.erf(x / _SQRT2))


def _gstage(x, s, t, w, b):
    return _gelu(jnp.dot(x * s + t, w, preferred_element_type=jnp.float32) + b)


def _l2norm(x):
    return x * lax.rsqrt(jnp.maximum(jnp.sum(x * x, axis=-1, keepdims=True), 1e-12))


def _fold(layer):
    s = layer["gamma"] * lax.rsqrt(layer["var"] + EPS)
    t = layer["beta"] - layer["mean"] * s
    return s.reshape(1, -1), t.reshape(1, -1), layer["W"], layer["b"].reshape(1, -1)


def _tc1(nf, ew2d, pre, prep1):
    def body(nf_r, ew_r,
             s0, t0, w0, b0, s1, t1, w1, b1,
             q0, r0, v0, c0, q1, r1, v1, c1,
             x0_o, p1_o, inv_o):
        x = nf_r[...]
        x = _gstage(x, s0[...], t0[...], w0[...], b0[...])
        x = _gstage(x, s1[...], t1[...], w1[...], b1[...])
        x0_o[...] = x
        p = _gstage(x, q0[...], r0[...], v0[...], c0[...])
        p = _gstage(p, q1[...], r1[...], v1[...], c1[...])
        p1_o[...] = p
        inv_o[...] = (1.0 / jnp.sum(ew_r[...])).reshape(1, 1)

    outs = [jax.ShapeDtypeStruct((N, H), jnp.float32),
            jax.ShapeDtypeStruct((N, H), jnp.float32),
            jax.ShapeDtypeStruct((1, 1), jnp.float32)]
    return pl.pallas_call(body, out_shape=outs)(
        nf, ew2d, *pre[0], *pre[1], *prep1[0], *prep1[1])


def _tc_update(x, pa, pb, inv_s, upd, nxt):
    (s0, t0, w0, b0), (s1, t1, w1, b1) = upd
    sxa, sxb = s0[:, :H], s0[:, H:]
    txa, txb = t0[:, :H], t0[:, H:]
    wa, wb = w0[:H], w0[H:]
    (m0s, m0t, m0w, m0b), (m1s, m1t, m1w, m1b) = nxt

    def body(x_r, pa_r, pb_r, inv_r,
             sxa_r, txa_r, sxb_r, txb_r, wa_r, wb_r, b0_r,
             s1_r, t1_r, w1_r, b1_r,
             a0s, a0t, a0w, a0b, a1s, a1t, a1w, a1b,
             xo, po):
        xx = x_r[...]
        agg = (pa_r[...] + pb_r[...]) * inv_r[0, 0]
        h = (jnp.dot(xx * sxa_r[...] + txa_r[...], wa_r[...],
                     preferred_element_type=jnp.float32)
             + jnp.dot(agg * sxb_r[...] + txb_r[...], wb_r[...],
                       preferred_element_type=jnp.float32)
             + b0_r[...])
        u = _gelu(h)
        u = _gstage(u, s1_r[...], t1_r[...], w1_r[...], b1_r[...])
        x_new = _l2norm(u) + xx
        xo[...] = x_new
        p = _gstage(x_new, a0s[...], a0t[...], a0w[...], a0b[...])
        p = _gstage(p, a1s[...], a1t[...], a1w[...], a1b[...])
        po[...] = p

    outs = [jax.ShapeDtypeStruct((N, H), jnp.float32),
            jax.ShapeDtypeStruct((N, H), jnp.float32)]
    return pl.pallas_call(body, out_shape=outs)(
        x, pa, pb, inv_s, sxa, txa, sxb, txb, wa, wb, b0,
        s1, t1, w1, b1, m0s, m0t, m0w, m0b, m1s, m1t, m1w, m1b)


def _tc_logits(emb, w, b):
    def body(e_r, w_r, b_r, o):
        o[...] = jnp.dot(e_r[...], w_r[...],
                         preferred_element_type=jnp.float32) + b_r[...]

    return pl.pallas_call(
        body, out_shape=jax.ShapeDtypeStruct((B, w.shape[1]), jnp.float32),
    )(emb, w, b)


def _sc_aggregate(prep, src2d, dst2d, ew2d, zrows):
    mesh = plsc.VectorSubcoreMesh(core_axis_name="c", subcore_axis_name="s")

    @functools.partial(
        pl.kernel,
        mesh=mesh,
        compiler_params=pltpu.CompilerParams(use_tc_tiling_on_sc=False),
        out_type=jax.ShapeDtypeStruct((NC, N_PAD, H), jnp.float32),
        scratch_types=[
            pltpu.VMEM((E_PER_W,), jnp.int32),
            pltpu.VMEM((E_PER_W,), jnp.int32),
            pltpu.VMEM((E_PER_W,), jnp.float32),
            pltpu.VMEM((CHUNK, H), jnp.float32),
            pltpu.VMEM((CHUNK, H), jnp.float32),
            pltpu.VMEM_SHARED((N_PAD, H), jnp.float32),
            pltpu.SemaphoreType.DMA,
            pltpu.SemaphoreType.DMA,
            pltpu.SemaphoreType.DMA,
            pltpu.SemaphoreType.DMA,
        ],
    )
    def k(prep_hbm, src_hbm, dst_hbm, ew_hbm, z_hbm, out_hbm,
          src_v, dst_v, ew_v, rows0_v, rows1_v, agg_sh, g0, g1, s0, s1):
        c = lax.axis_index("c")
        s = lax.axis_index("s")
        w = c * NS + s
        pltpu.sync_copy(z_hbm, agg_sh.at[pl.ds(s * ROWS_PER_TILE, ROWS_PER_TILE)])
        pltpu.sync_copy(src_hbm.at[pl.ds(w * E_PER_W, E_PER_W)], src_v)
        pltpu.sync_copy(dst_hbm.at[pl.ds(w * E_PER_W, E_PER_W)], dst_v)
        pltpu.sync_copy(ew_hbm.at[pl.ds(w * E_PER_W, E_PER_W)], ew_v)
        plsc.subcore_barrier()

        def scale(rows_v, t):
            for g in range(CHUNK // L):
                ewg = ew_v[pl.ds(t * CHUNK + g * L, L)]
                for j in range(L):
                    e = g * L + j
                    splat = lax.gather(
                        ewg, jnp.full((L, 1), j, jnp.int32),
                        lax.GatherDimensionNumbers(
                            offset_dims=(), collapsed_slice_dims=(0,),
                            start_index_map=(0,)),
                        slice_sizes=(1,),
                        mode=lax.GatherScatterMode.PROMISE_IN_BOUNDS)
                    rows_v[e, pl.ds(0, L)] = rows_v[e, pl.ds(0, L)] * splat
                    rows_v[e, pl.ds(L, L)] = rows_v[e, pl.ds(L, L)] * splat

        def start_gather(t, buf, sem):
            pltpu.async_copy(prep_hbm.at[src_v.at[pl.ds(t * CHUNK, CHUNK)]], buf, sem)

        def wait_gather(t, buf, sem):
            pltpu.make_async_copy(prep_hbm.at[src_v.at[pl.ds(t * CHUNK, CHUNK)]], buf, sem).wait()

        start_gather(0, rows0_v, g0)

        def pair_body(i, carry):
            t0 = 2 * i
            t1 = t0 + 1
            start_gather(t1, rows1_v, g1)
            wait_gather(t0, rows0_v, g0)
            scale(rows0_v, t0)
            h0 = pltpu.async_copy(rows0_v, agg_sh.at[dst_v.at[pl.ds(t0 * CHUNK, CHUNK)]], s0,
                                  add=True)
            wait_gather(t1, rows1_v, g1)
            scale(rows1_v, t1)
            h1 = pltpu.async_copy(rows1_v, agg_sh.at[dst_v.at[pl.ds(t1 * CHUNK, CHUNK)]], s1,
                                  add=True)
            h0.wait()

            @pl.when(t0 + 2 < N_CHUNKS)
            def _():
                start_gather(t0 + 2, rows0_v, g0)

            h1.wait()
            return carry

        lax.fori_loop(0, N_CHUNKS // 2, pair_body, 0)
        plsc.subcore_barrier()
        pltpu.sync_copy(
            agg_sh.at[pl.ds(s * ROWS_PER_TILE, ROWS_PER_TILE)],
            out_hbm.at[c, pl.ds(s * ROWS_PER_TILE, ROWS_PER_TILE)])

    return k(prep, src2d, dst2d, ew2d, zrows)


def _sc_gather_rows(table, idx):
    b_per_w = B // NW
    mesh = plsc.VectorSubcoreMesh(core_axis_name="c", subcore_axis_name="s")

    @functools.partial(
        pl.kernel,
        mesh=mesh,
        compiler_params=pltpu.CompilerParams(use_tc_tiling_on_sc=False),
        out_type=jax.ShapeDtypeStruct((B, H), jnp.float32),
        scratch_types=[
            pltpu.VMEM((b_per_w,), jnp.int32),
            pltpu.VMEM((b_per_w, H), jnp.float32),
            pltpu.SemaphoreType.DMA,
        ],
    )
    def k(table_hbm, idx_hbm, out_hbm, idx_v, rows_v, sem):
        w = lax.axis_index("s") * NC + lax.axis_index("c")
        base = w * b_per_w
        pltpu.sync_copy(idx_hbm.at[pl.ds(base, b_per_w)], idx_v)
        pltpu.async_copy(table_hbm.at[idx_v], rows_v, sem).wait()
        pltpu.sync_copy(rows_v, out_hbm.at[pl.ds(base, b_per_w)])

    return k(table, idx)


def kernel(node_features, edge_weights, params, edges, input_node_indices):
    p = params
    pre = [_fold(l) for l in p["preprocess"]]
    prep1 = [_fold(l) for l in p["conv1_prepare"]]
    upd1 = [_fold(l) for l in p["conv1_update"]]
    prep2 = [_fold(l) for l in p["conv2_prepare"]]
    upd2 = [_fold(l) for l in p["conv2_update"]]
    post = [_fold(l) for l in p["postprocess"]]

    ew_sum2d = edge_weights.reshape(E // DF, DF)
    src1d = edges[1]
    dst1d = edges[0]

    zrows = jnp.zeros((ROWS_PER_TILE, H), jnp.float32)

    x0, prep1_t, inv_s = _tc1(node_features, ew_sum2d, pre, prep1)
    part1 = _sc_aggregate(prep1_t, src1d, dst1d, edge_weights, zrows)
    x1, prep2_t = _tc_update(x0, part1[0, :N], part1[1, :N], inv_s, upd1, prep2)
    part2 = _sc_aggregate(prep2_t, src1d, dst1d, edge_weights, zrows)
    _, y = _tc_update(x1, part2[0, :N], part2[1, :N], inv_s, upd2, post)
    emb = _sc_gather_rows(y, input_node_indices)
    return _tc_logits(emb, p["logits_W"], p["logits_b"].reshape(1, -1))


# speedup vs baseline: 2.0589x; 1.2490x over previous
import functools

import jax
import jax.numpy as jnp
from jax import lax
from jax.experimental import pallas as pl
from jax.experimental.pallas import tpu as pltpu
from jax.experimental.pallas import tpu_sc as plsc

N = 10000
DF = 128
H = 32
E = 640000
B = 1024
EPS = 1e-3

NC, NS, L = 2, 16, 16
NW = NC * NS
CHUNK = 80
E_PER_W = E // NW
N_CHUNKS = E_PER_W // CHUNK
NBUF = 5
N_PAD = 10112
ROWS_PER_TILE = N_PAD // NS

_SQRT2 = 1.4142135623730951


def _gelu(x):
    return 0.5 * x * (1.0 + lax.erf(x / _SQRT2))


def _gstage(x, s, t, w, b):
    return _gelu(jnp.dot(x * s + t, w, preferred_element_type=jnp.float32) + b)


def _l2norm(x):
    return x * lax.rsqrt(jnp.maximum(jnp.sum(x * x, axis=-1, keepdims=True), 1e-12))


def _fold(layer):
    s = layer["gamma"] * lax.rsqrt(layer["var"] + EPS)
    t = layer["beta"] - layer["mean"] * s
    return s.reshape(1, -1), t.reshape(1, -1), layer["W"], layer["b"].reshape(1, -1)


def _tc1(nf, ew2d, pre, prep1):
    def body(nf_r, ew_r,
             s0, t0, w0, b0, s1, t1, w1, b1,
             q0, r0, v0, c0, q1, r1, v1, c1,
             x0_o, p1_o, inv_o):
        x = nf_r[...]
        x = _gstage(x, s0[...], t0[...], w0[...], b0[...])
        x = _gstage(x, s1[...], t1[...], w1[...], b1[...])
        x0_o[...] = x
        p = _gstage(x, q0[...], r0[...], v0[...], c0[...])
        p = _gstage(p, q1[...], r1[...], v1[...], c1[...])
        p1_o[...] = p
        inv_o[...] = (1.0 / jnp.sum(ew_r[...])).reshape(1, 1)

    outs = [jax.ShapeDtypeStruct((N, H), jnp.float32),
            jax.ShapeDtypeStruct((N, H), jnp.float32),
            jax.ShapeDtypeStruct((1, 1), jnp.float32)]
    return pl.pallas_call(body, out_shape=outs)(
        nf, ew2d, *pre[0], *pre[1], *prep1[0], *prep1[1])


def _tc_update(x, pa, pb, inv_s, upd, nxt):
    (s0, t0, w0, b0), (s1, t1, w1, b1) = upd
    sxa, sxb = s0[:, :H], s0[:, H:]
    txa, txb = t0[:, :H], t0[:, H:]
    wa, wb = w0[:H], w0[H:]
    (m0s, m0t, m0w, m0b), (m1s, m1t, m1w, m1b) = nxt

    def body(x_r, pa_r, pb_r, inv_r,
             sxa_r, txa_r, sxb_r, txb_r, wa_r, wb_r, b0_r,
             s1_r, t1_r, w1_r, b1_r,
             a0s, a0t, a0w, a0b, a1s, a1t, a1w, a1b,
             xo, po):
        xx = x_r[...]
        agg = (pa_r[...] + pb_r[...]) * inv_r[0, 0]
        h = (jnp.dot(xx * sxa_r[...] + txa_r[...], wa_r[...],
                     preferred_element_type=jnp.float32)
             + jnp.dot(agg * sxb_r[...] + txb_r[...], wb_r[...],
                       preferred_element_type=jnp.float32)
             + b0_r[...])
        u = _gelu(h)
        u = _gstage(u, s1_r[...], t1_r[...], w1_r[...], b1_r[...])
        x_new = _l2norm(u) + xx
        xo[...] = x_new
        p = _gstage(x_new, a0s[...], a0t[...], a0w[...], a0b[...])
        p = _gstage(p, a1s[...], a1t[...], a1w[...], a1b[...])
        po[...] = p

    outs = [jax.ShapeDtypeStruct((N, H), jnp.float32),
            jax.ShapeDtypeStruct((N, H), jnp.float32)]
    return pl.pallas_call(body, out_shape=outs)(
        x, pa, pb, inv_s, sxa, txa, sxb, txb, wa, wb, b0,
        s1, t1, w1, b1, m0s, m0t, m0w, m0b, m1s, m1t, m1w, m1b)


def _tc_logits(emb, w, b):
    def body(e_r, w_r, b_r, o):
        o[...] = jnp.dot(e_r[...], w_r[...],
                         preferred_element_type=jnp.float32) + b_r[...]

    return pl.pallas_call(
        body, out_shape=jax.ShapeDtypeStruct((B, w.shape[1]), jnp.float32),
    )(emb, w, b)


def _sc_aggregate(prep, src2d, dst2d, ew2d, zrows):
    mesh = plsc.VectorSubcoreMesh(core_axis_name="c", subcore_axis_name="s")

    @functools.partial(
        pl.kernel,
        mesh=mesh,
        compiler_params=pltpu.CompilerParams(use_tc_tiling_on_sc=False),
        out_type=jax.ShapeDtypeStruct((NC, N_PAD, H), jnp.float32),
        scratch_types=[
            pltpu.VMEM((E_PER_W,), jnp.int32),
            pltpu.VMEM((E_PER_W,), jnp.int32),
            pltpu.VMEM((E_PER_W,), jnp.float32),
            [pltpu.VMEM((CHUNK, H), jnp.float32)] * NBUF,
            pltpu.VMEM_SHARED((N_PAD, H), jnp.float32),
            [pltpu.SemaphoreType.DMA] * NBUF,
            [pltpu.SemaphoreType.DMA] * NBUF,
        ],
    )
    def k(prep_hbm, src_hbm, dst_hbm, ew_hbm, z_hbm, out_hbm,
          src_v, dst_v, ew_v, rows_bufs, agg_sh, gsems, ssems):
        c = lax.axis_index("c")
        s = lax.axis_index("s")
        w = c * NS + s
        pltpu.sync_copy(z_hbm, agg_sh.at[pl.ds(s * ROWS_PER_TILE, ROWS_PER_TILE)])
        pltpu.sync_copy(src_hbm.at[pl.ds(w * E_PER_W, E_PER_W)], src_v)
        pltpu.sync_copy(dst_hbm.at[pl.ds(w * E_PER_W, E_PER_W)], dst_v)
        pltpu.sync_copy(ew_hbm.at[pl.ds(w * E_PER_W, E_PER_W)], ew_v)
        plsc.subcore_barrier()

        def scale(rows_v, t):
            for g in range(CHUNK // L):
                ewg = ew_v[pl.ds(t * CHUNK + g * L, L)]
                for j in range(L):
                    e = g * L + j
                    splat = lax.gather(
                        ewg, jnp.full((L, 1), j, jnp.int32),
                        lax.GatherDimensionNumbers(
                            offset_dims=(), collapsed_slice_dims=(0,),
                            start_index_map=(0,)),
                        slice_sizes=(1,),
                        mode=lax.GatherScatterMode.PROMISE_IN_BOUNDS)
                    rows_v[e, pl.ds(0, L)] = rows_v[e, pl.ds(0, L)] * splat
                    rows_v[e, pl.ds(L, L)] = rows_v[e, pl.ds(L, L)] * splat

        def start_gather(t, buf, sem):
            pltpu.async_copy(prep_hbm.at[src_v.at[pl.ds(t * CHUNK, CHUNK)]], buf, sem)

        def wait_gather(t, buf, sem):
            pltpu.make_async_copy(prep_hbm.at[src_v.at[pl.ds(t * CHUNK, CHUNK)]], buf, sem).wait()

        for k2 in range(NBUF):
            start_gather(k2, rows_bufs[k2], gsems[k2])

        def group_body(i, carry):
            t0 = NBUF * i
            hs = []
            for k2 in range(NBUF):
                t = t0 + k2
                wait_gather(t, rows_bufs[k2], gsems[k2])
                scale(rows_bufs[k2], t)
                hs.append(pltpu.async_copy(
                    rows_bufs[k2],
                    agg_sh.at[dst_v.at[pl.ds(t * CHUNK, CHUNK)]],
                    ssems[k2], add=True))
            for h in hs:
                h.wait()

            @pl.when(t0 + NBUF < N_CHUNKS)
            def _():
                for k2 in range(NBUF):
                    start_gather(t0 + NBUF + k2, rows_bufs[k2], gsems[k2])

            return carry

        lax.fori_loop(0, N_CHUNKS // NBUF, group_body, 0)
        plsc.subcore_barrier()
        pltpu.sync_copy(
            agg_sh.at[pl.ds(s * ROWS_PER_TILE, ROWS_PER_TILE)],
            out_hbm.at[c, pl.ds(s * ROWS_PER_TILE, ROWS_PER_TILE)])

    return k(prep, src2d, dst2d, ew2d, zrows)


def _sc_gather_rows(table, idx):
    b_per_w = B // NW
    mesh = plsc.VectorSubcoreMesh(core_axis_name="c", subcore_axis_name="s")

    @functools.partial(
        pl.kernel,
        mesh=mesh,
        compiler_params=pltpu.CompilerParams(use_tc_tiling_on_sc=False),
        out_type=jax.ShapeDtypeStruct((B, H), jnp.float32),
        scratch_types=[
            pltpu.VMEM((b_per_w,), jnp.int32),
            pltpu.VMEM((b_per_w, H), jnp.float32),
            pltpu.SemaphoreType.DMA,
        ],
    )
    def k(table_hbm, idx_hbm, out_hbm, idx_v, rows_v, sem):
        w = lax.axis_index("s") * NC + lax.axis_index("c")
        base = w * b_per_w
        pltpu.sync_copy(idx_hbm.at[pl.ds(base, b_per_w)], idx_v)
        pltpu.async_copy(table_hbm.at[idx_v], rows_v, sem).wait()
        pltpu.sync_copy(rows_v, out_hbm.at[pl.ds(base, b_per_w)])

    return k(table, idx)


def kernel(node_features, edge_weights, params, edges, input_node_indices):
    p = params
    pre = [_fold(l) for l in p["preprocess"]]
    prep1 = [_fold(l) for l in p["conv1_prepare"]]
    upd1 = [_fold(l) for l in p["conv1_update"]]
    prep2 = [_fold(l) for l in p["conv2_prepare"]]
    upd2 = [_fold(l) for l in p["conv2_update"]]
    post = [_fold(l) for l in p["postprocess"]]

    ew_sum2d = edge_weights.reshape(E // DF, DF)
    src1d = edges[1]
    dst1d = edges[0]

    zrows = jnp.zeros((ROWS_PER_TILE, H), jnp.float32)

    x0, prep1_t, inv_s = _tc1(node_features, ew_sum2d, pre, prep1)
    part1 = _sc_aggregate(prep1_t, src1d, dst1d, edge_weights, zrows)
    x1, prep2_t = _tc_update(x0, part1[0, :N], part1[1, :N], inv_s, upd1, prep2)
    part2 = _sc_aggregate(prep2_t, src1d, dst1d, edge_weights, zrows)
    _, y = _tc_update(x1, part2[0, :N], part2[1, :N], inv_s, upd2, post)
    emb = _sc_gather_rows(y, input_node_indices)
    return _tc_logits(emb, p["logits_W"], p["logits_b"].reshape(1, -1))
